# baseline (speedup 1.0000x reference)
import jax
import jax.numpy as jnp
from jax import lax
from jax.experimental import pallas as pl
from jax.experimental.pallas import tpu as pltpu
from jax.experimental.pallas import tpu_sc as plsc

B = 4096
F = 26
V = 100000
D = 32
ND = 13
OUT = F * D + ND

_info = plsc.get_sparse_core_info()
NC, NS, NL = _info.num_cores, _info.num_subcores, _info.num_lanes
NW = NC * NS
BPW = B // NW


def _body(sparse_r, dense, table, out, idx_v, rows_v, dense_v, sem):
    wid = lax.axis_index("s") * NC + lax.axis_index("c")
    base = wid * BPW

    pltpu.sync_copy(sparse_r.at[wid], idx_v)

    for f in range(1, F):
        off = f * (V + 1)
        for j in range(BPW // NL):
            sl = pl.ds(j * NL, NL)
            idx_v[f, sl] = idx_v[f, sl] + off

    for f0 in range(0, F, 13):
        cps = [
            pltpu.async_copy(table.at[idx_v.at[f]], rows_v.at[f], sem)
            for f in range(f0, f0 + 13)
        ]
        for cp in cps:
            cp.wait()

    pltpu.sync_copy(dense.at[pl.ds(base, BPW), :], dense_v)

    for f in range(F):
        pltpu.sync_copy(rows_v.at[f], out.at[pl.ds(base, BPW), pl.ds(f * D, D)])
    pltpu.sync_copy(dense_v, out.at[pl.ds(base, BPW), pl.ds(F * D, ND)])


def kernel(dense_x, sparse_x, tables):
    dense_x = dense_x.astype(jnp.float32)
    table_flat = tables.astype(jnp.float32).reshape(F * (V + 1), D)
    sparse_r = (
        sparse_x.astype(jnp.int32).T.reshape(F, NW, BPW).transpose(1, 0, 2)
    )

    k = pl.kernel(
        _body,
        out_type=jax.ShapeDtypeStruct((B, OUT), jnp.float32),
        mesh=plsc.VectorSubcoreMesh(core_axis_name="c", subcore_axis_name="s"),
        scratch_types=[
            pltpu.VMEM((F, BPW), jnp.int32),
            pltpu.VMEM((F, BPW, D), jnp.float32),
            pltpu.VMEM((BPW, ND), jnp.float32),
            pltpu.SemaphoreType.DMA,
        ],
        compiler_params=pltpu.CompilerParams(use_tc_tiling_on_sc=False),
    )
    return k(sparse_r, dense_x, table_flat)


# speedup vs baseline: 3.0290x; 3.0290x over previous
import jax
import jax.numpy as jnp
from jax import lax
from jax.experimental import pallas as pl
from jax.experimental.pallas import tpu as pltpu
from jax.experimental.pallas import tpu_sc as plsc

B = 4096
F = 26
V = 100000
D = 32
ND = 13
OUT = F * D + ND

_info = plsc.get_sparse_core_info()
NC, NS, NL = _info.num_cores, _info.num_subcores, _info.num_lanes
NW = NC * NS


def _body(sparse_flat, dense_t, tables_e, out_t, idx_v, g0, g1, dv, gsem, wsem):
    wid = lax.axis_index("s") * NC + lax.axis_index("c")

    bufs = (g0, g1)
    wcp = [None, None]
    for s in range(F):
        b = s % 2
        pltpu.sync_copy(sparse_flat.at[pl.ds(s * B, B)], idx_v)
        if wcp[b] is not None:
            wcp[b].wait()
        pltpu.async_copy(
            tables_e.at[s, wid].at[idx_v], bufs[b], gsem
        ).wait()
        wcp[b] = pltpu.async_copy(
            bufs[b], out_t.at[s * D + wid], wsem
        )

    @pl.when(wid < ND)
    def _():
        pltpu.sync_copy(dense_t.at[wid], dv)
        pltpu.sync_copy(dv, out_t.at[F * D + wid])

    for cp in wcp:
        if cp is not None:
            cp.wait()


def kernel(dense_x, sparse_x, tables):
    tables_e = jnp.transpose(tables.astype(jnp.float32), (0, 2, 1))
    sparse_flat = sparse_x.astype(jnp.int32).T.reshape(F * B)
    dense_t = dense_x.astype(jnp.float32).T

    k = pl.kernel(
        _body,
        out_type=jax.ShapeDtypeStruct((OUT, B), jnp.float32),
        mesh=plsc.VectorSubcoreMesh(core_axis_name="c", subcore_axis_name="s"),
        scratch_types=[
            pltpu.VMEM((B,), jnp.int32),
            pltpu.VMEM((B,), jnp.float32),
            pltpu.VMEM((B,), jnp.float32),
            pltpu.VMEM((B,), jnp.float32),
            pltpu.SemaphoreType.DMA,
            pltpu.SemaphoreType.DMA,
        ],
        compiler_params=pltpu.CompilerParams(use_tc_tiling_on_sc=False),
    )
    out_t = k(sparse_flat, dense_t, tables_e)
    return out_t.T


# speedup vs baseline: 20.0098x; 6.6062x over previous
import jax
import jax.numpy as jnp
from jax import lax
from jax.experimental import pallas as pl
from jax.experimental.pallas import tpu as pltpu
from jax.experimental.pallas import tpu_sc as plsc

B = 4096
F = 26
V = 100000
D = 32
ND = 13
OUT = F * D + ND

_info = plsc.get_sparse_core_info()
NC, NS, NL = _info.num_cores, _info.num_subcores, _info.num_lanes
NW = NC * NS


def _body(sparse_flat, dense_t, tables_e, out_t, idx_v, g0, g1, dv, gsem, wsem):
    wid = lax.axis_index("s") * NC + lax.axis_index("c")

    bufs = (g0, g1)
    wcp = [None, None]
    for s in range(F):
        b = s % 2
        pltpu.sync_copy(sparse_flat.at[pl.ds(s * B, B)], idx_v)
        if wcp[b] is not None:
            wcp[b].wait()
        pltpu.async_copy(
            tables_e.at[s, wid].at[idx_v], bufs[b], gsem
        ).wait()
        wcp[b] = pltpu.async_copy(
            bufs[b], out_t.at[s * D + wid], wsem
        )

    @pl.when(wid < ND)
    def _():
        pltpu.sync_copy(dense_t.at[wid], dv)
        pltpu.sync_copy(dv, out_t.at[F * D + wid])

    for cp in wcp:
        if cp is not None:
            cp.wait()


def kernel(dense_x, sparse_x, tables):
    tables_e = jnp.pad(jnp.transpose(tables.astype(jnp.float32), (0, 2, 1)), ((0, 0), (0, 0), (0, 95)))
    sparse_flat = sparse_x.astype(jnp.int32).T.reshape(F * B)
    dense_t = dense_x.astype(jnp.float32).T

    k = pl.kernel(
        _body,
        out_type=jax.ShapeDtypeStruct((OUT, B), jnp.float32),
        mesh=plsc.VectorSubcoreMesh(core_axis_name="c", subcore_axis_name="s"),
        scratch_types=[
            pltpu.VMEM((B,), jnp.int32),
            pltpu.VMEM((B,), jnp.float32),
            pltpu.VMEM((B,), jnp.float32),
            pltpu.VMEM((B,), jnp.float32),
            pltpu.SemaphoreType.DMA,
            pltpu.SemaphoreType.DMA,
        ],
        compiler_params=pltpu.CompilerParams(use_tc_tiling_on_sc=False),
    )
    out_t = k(sparse_flat, dense_t, tables_e)
    return out_t.T


# speedup vs baseline: 21.3220x; 1.0656x over previous
import jax
import jax.numpy as jnp
from jax import lax
from jax.experimental import pallas as pl
from jax.experimental.pallas import tpu as pltpu
from jax.experimental.pallas import tpu_sc as plsc

B = 4096
F = 26
V = 100000
D = 32
ND = 13
OUT = F * D + ND

_info = plsc.get_sparse_core_info()
NC, NS, NL = _info.num_cores, _info.num_subcores, _info.num_lanes
NW = NC * NS


def _body(sparse_flat, dense_t, tables_e, out_t, idx0, idx1, r0, r1, dv,
          g0, g1, wsem):
    wid = lax.axis_index("s") * NC + lax.axis_index("c")

    idxs = (idx0, idx1)
    rows = (r0, r1)
    gsems = (g0, g1)
    gcp = [None, None]
    wcp = [None, None]

    pltpu.sync_copy(sparse_flat.at[pl.ds(0, B)], idxs[0])
    gcp[0] = pltpu.async_copy(tables_e.at[0, wid].at[idxs[0]], rows[0], gsems[0])
    for s in range(1, F + 1):
        b = s % 2
        if s < F:
            pltpu.sync_copy(sparse_flat.at[pl.ds(s * B, B)], idxs[b])
            if wcp[b] is not None:
                wcp[b].wait()
            gcp[b] = pltpu.async_copy(
                tables_e.at[s, wid].at[idxs[b]], rows[b], gsems[b]
            )
        pb = (s - 1) % 2
        gcp[pb].wait()
        wcp[pb] = pltpu.async_copy(rows[pb], out_t.at[(s - 1) * D + wid], wsem)

    @pl.when(wid < ND)
    def _():
        pltpu.sync_copy(dense_t.at[wid], dv)
        pltpu.sync_copy(dv, out_t.at[F * D + wid])

    for cp in wcp:
        if cp is not None:
            cp.wait()


def kernel(dense_x, sparse_x, tables):
    tables_e = jnp.pad(jnp.transpose(tables.astype(jnp.float32), (0, 2, 1)), ((0, 0), (0, 0), (0, 95)))
    sparse_flat = sparse_x.astype(jnp.int32).T.reshape(F * B)
    dense_t = dense_x.astype(jnp.float32).T

    k = pl.kernel(
        _body,
        out_type=jax.ShapeDtypeStruct((OUT, B), jnp.float32),
        mesh=plsc.VectorSubcoreMesh(core_axis_name="c", subcore_axis_name="s"),
        scratch_types=[
            pltpu.VMEM((B,), jnp.int32),
            pltpu.VMEM((B,), jnp.int32),
            pltpu.VMEM((B,), jnp.float32),
            pltpu.VMEM((B,), jnp.float32),
            pltpu.VMEM((B,), jnp.float32),
            pltpu.SemaphoreType.DMA,
            pltpu.SemaphoreType.DMA,
            pltpu.SemaphoreType.DMA,
        ],
        compiler_params=pltpu.CompilerParams(use_tc_tiling_on_sc=False),
    )
    out_t = k(sparse_flat, dense_t, tables_e)
    return out_t.T
